# Initial kernel scaffold; baseline (speedup 1.0000x reference)
import jax
import jax.numpy as jnp
from jax.experimental import pallas as pl


def kernel(X, edge_index, edge_weight, W_band, b_band, W_fuse, b_fuse):
    raise NotImplementedError("write your pallas kernel here")


# speedup vs baseline: 3.1693x; 3.1693x over previous
import functools
import math

import jax
import jax.numpy as jnp
import numpy as np
from jax import lax
from jax.experimental import pallas as pl
from jax.experimental.pallas import tpu as pltpu
from jax.experimental.pallas import tpu_sc as plsc

_K = 8
_TAUS = [0.5, 1.5, 4.0]
_M = len(_TAUS)


def _bessel_i(k, x):
    s = 0.0
    for m in range(40):
        s += (0.5 * x) ** (2 * m + k) / (math.factorial(m) * math.factorial(m + k))
    return s


def _cheb_coeffs(tau, K):
    a = np.zeros(K + 1, dtype=np.float64)
    if tau == 0.0:
        a[0] = 1.0
        return a
    e = math.exp(-tau)
    a[0] = e * _bessel_i(0, tau)
    for k in range(1, K + 1):
        a[k] = 2.0 * e * ((-1.0) ** k) * _bessel_i(k, tau)
    return a


_A = np.stack([_cheb_coeffs(t, _K) for t in [0.0] + _TAUS], axis=0)


def _make_spmm(N, D, E):
    NC, NS = 2, 16
    NW = NC * NS
    EPW = E // NW
    CHUNK = 80
    NCH = EPW // CHUNK
    RPT = -(-N // NS // 8) * 8
    RPT_TAIL = N - (NS - 1) * RPT
    LANES = 16

    mesh = plsc.VectorSubcoreMesh(core_axis_name="c", subcore_axis_name="s")

    @functools.partial(
        pl.kernel,
        out_type=jax.ShapeDtypeStruct((NC, N, D), jnp.float32),
        mesh=mesh,
        scratch_types=[
            pltpu.VMEM((CHUNK,), jnp.int32),
            pltpu.VMEM((CHUNK,), jnp.int32),
            pltpu.VMEM((CHUNK, LANES), jnp.float32),
            pltpu.VMEM((CHUNK, D), jnp.float32),
            pltpu.VMEM_SHARED((N, D), jnp.float32),
            pltpu.SemaphoreType.DMA,
        ],
    )
    def spmm(x_hbm, rows_hbm, cols_hbm, w_hbm, zero_hbm, out_hbm,
             colv, rowv, wv, gbuf, acc, sem):
        cid = lax.axis_index("c")
        sid = lax.axis_index("s")
        wid = sid * NC + cid

        rbase = pl.multiple_of(sid * RPT, 8)

        @pl.when(sid < NS - 1)
        def _():
            pltpu.sync_copy(zero_hbm.at[pl.ds(rbase, RPT)],
                            acc.at[pl.ds(rbase, RPT)])

        @pl.when(sid == NS - 1)
        def _():
            pltpu.sync_copy(zero_hbm.at[pl.ds((NS - 1) * RPT, RPT_TAIL)],
                            acc.at[pl.ds((NS - 1) * RPT, RPT_TAIL)])

        plsc.subcore_barrier()

        base0 = wid * EPW

        def chunk_body(ci, carry):
            base = pl.multiple_of(base0 + ci * CHUNK, 8)
            pltpu.sync_copy(cols_hbm.at[pl.ds(base, CHUNK)], colv)
            pltpu.sync_copy(rows_hbm.at[pl.ds(base, CHUNK)], rowv)
            pltpu.sync_copy(w_hbm.at[pl.ds(base, CHUNK)], wv)
            pltpu.async_copy(x_hbm.at[colv], gbuf, sem).wait()

            def edge_body(ei, c2):
                wsplat = wv[ei, :]
                for j in range(D // LANES):
                    seg = gbuf[ei, pl.ds(j * LANES, LANES)]
                    gbuf[ei, pl.ds(j * LANES, LANES)] = seg * wsplat
                return c2

            lax.fori_loop(0, CHUNK, edge_body, 0, unroll=False)
            pltpu.sync_copy(gbuf, acc.at[rowv], add=True)
            return carry

        lax.fori_loop(0, NCH, chunk_body, 0, unroll=False)

        plsc.subcore_barrier()

        @pl.when(sid < NS - 1)
        def _():
            pltpu.sync_copy(acc.at[pl.ds(rbase, RPT)],
                            out_hbm.at[cid, pl.ds(rbase, RPT)])

        @pl.when(sid == NS - 1)
        def _():
            pltpu.sync_copy(acc.at[pl.ds((NS - 1) * RPT, RPT_TAIL)],
                            out_hbm.at[cid, pl.ds((NS - 1) * RPT, RPT_TAIL)])

    return spmm


def _combine(p, prev, scale, sub):
    N, D = prev.shape
    BN = 1000

    def body(p_ref, prev_ref, o_ref):
        s = p_ref[0] + p_ref[1]
        o_ref[...] = scale * s - sub * prev_ref[...]

    return pl.pallas_call(
        body,
        grid=(N // BN,),
        in_specs=[
            pl.BlockSpec((2, BN, D), lambda i: (0, i, 0)),
            pl.BlockSpec((BN, D), lambda i: (i, 0)),
        ],
        out_specs=pl.BlockSpec((BN, D), lambda i: (i, 0)),
        out_shape=jax.ShapeDtypeStruct((N, D), jnp.float32),
    )(p, prev)


def _tail(psis, X, W_band, b_band, W_fuse, b_fuse):
    N, D = X.shape
    BN = 1000
    KP1 = len(psis)
    A = _A.astype(np.float32)

    def body(*refs):
        psi_refs = refs[:KP1]
        x_ref, wb_ref, bb_ref, wf_ref, bf_ref, o_ref = refs[KP1:]
        psi = [r[...] for r in psi_refs]
        Y = []
        for i in range(_M + 1):
            acc = float(A[i, 0]) * psi[0]
            for k in range(1, KP1):
                acc = acc + float(A[i, k]) * psi[k]
            Y.append(acc)
        wf = wf_ref[...]
        z = jnp.zeros((BN, D), jnp.float32) + bf_ref[0]
        for i in range(1, _M + 1):
            band = Y[i - 1] - Y[i]
            h = jnp.maximum(
                jnp.dot(band, wb_ref[i - 1],
                        preferred_element_type=jnp.float32) + bb_ref[i - 1],
                0.0)
            z = z + jnp.dot(h, wf[(i - 1) * D:i * D],
                            preferred_element_type=jnp.float32)
        h0 = jnp.maximum(
            jnp.dot(Y[_M], wb_ref[_M],
                    preferred_element_type=jnp.float32) + bb_ref[_M],
            0.0)
        z = z + jnp.dot(h0, wf[_M * D:(_M + 1) * D],
                        preferred_element_type=jnp.float32)
        z = z + jnp.dot(x_ref[...], wf[(_M + 1) * D:(_M + 2) * D],
                        preferred_element_type=jnp.float32)
        o_ref[...] = z

    FIN = (_M + 2) * D
    in_specs = (
        [pl.BlockSpec((BN, D), lambda i: (i, 0)) for _ in range(KP1)]
        + [
            pl.BlockSpec((BN, D), lambda i: (i, 0)),
            pl.BlockSpec((_M + 1, D, D), lambda i: (0, 0, 0)),
            pl.BlockSpec((_M + 1, D), lambda i: (0, 0)),
            pl.BlockSpec((FIN, D), lambda i: (0, 0)),
            pl.BlockSpec((1, D), lambda i: (0, 0)),
        ]
    )
    return pl.pallas_call(
        body,
        grid=(N // BN,),
        in_specs=in_specs,
        out_specs=pl.BlockSpec((BN, D), lambda i: (i, 0)),
        out_shape=jax.ShapeDtypeStruct((N, D), jnp.float32),
    )(*psis, X, W_band, b_band, W_fuse, b_fuse.reshape(1, D))


def kernel(X, edge_index, edge_weight, W_band, b_band, W_fuse, b_fuse):
    N, D = X.shape
    E = edge_weight.shape[0]
    rows = edge_index[0]
    cols = edge_index[1]
    w16 = jnp.broadcast_to(edge_weight[:, None], (E, 16))
    zero = jnp.zeros((N, D), jnp.float32)

    spmm = _make_spmm(N, D, E)

    psis = [X]
    p = spmm(X, rows, cols, w16, zero)
    psis.append(_combine(p, X, 1.0, 0.0))
    for _ in range(2, _K + 1):
        p = spmm(psis[-1], rows, cols, w16, zero)
        psis.append(_combine(p, psis[-2], 2.0, 1.0))

    return _tail(psis, X, W_band, b_band, W_fuse, b_fuse)
